# Initial kernel scaffold; baseline (speedup 1.0000x reference)
import jax
import jax.numpy as jnp
from jax.experimental import pallas as pl


def kernel(prev_action, action_emb_weight):
    raise NotImplementedError("write your pallas kernel here")


# speedup vs baseline: 6.6471x; 6.6471x over previous
import jax
import jax.numpy as jnp
from jax.experimental import pallas as pl
from jax.experimental.pallas import tpu as pltpu
from jax.experimental.pallas import tpu_sc as plsc

BATCH = 16384
HIST = 200
EMB = 32
N = BATCH * HIST
WINDOW = 128


def _sc_gather(table_hbm, idx_hbm, out_hbm):
    def body(i_vmem, o_vmem):
        pltpu.sync_copy(table_hbm.at[i_vmem.at[0]], o_vmem)

    pltpu.emit_pipeline(
        body,
        grid=(N // WINDOW,),
        in_specs=[pl.BlockSpec((1, WINDOW), index_map=lambda i: (0, i))],
        out_specs=[pl.BlockSpec((WINDOW, EMB), index_map=lambda i: (i, 0))],
        core_axis_name=("c", "s"),
        dimension_semantics=(pltpu.PARALLEL,),
    )(idx_hbm, out_hbm)


@jax.jit
def kernel(prev_action, action_emb_weight):
    idx = prev_action.reshape(1, N).astype(jnp.int32)
    mesh = plsc.VectorSubcoreMesh(core_axis_name="c", subcore_axis_name="s")
    out = pl.kernel(
        _sc_gather,
        out_type=jax.ShapeDtypeStruct((N, EMB), jnp.float32),
        mesh=mesh,
        compiler_params=pltpu.CompilerParams(use_tc_tiling_on_sc=False),
    )(action_emb_weight, idx)
    return out.reshape(BATCH, HIST, EMB)
